# baseline (speedup 1.0000x reference)
import functools

import jax
import jax.numpy as jnp
from jax import lax
from jax.experimental import pallas as pl
from jax.experimental.pallas import tpu as pltpu
from jax.experimental.pallas import tpu_sc as plsc

_N = 10000
_E = 320000
_D = 128
_H = 128
_HID = 64
_C = 10
_G = 256

_NP = 10240
_NC = 2
_NS = 16
_NW = _NC * _NS
_RPT = _NP // _NS

_EP = 327680
_KE = 64
_NCHE = (_EP // _NW) // _KE

_KP = 64
_NCHP = 8
_PP = _NW * _NCHP * _KP

_CROWS = 512

_RB = 256


def _segment_rows_sc(t, src2, dst2, zrows, nch, k):
    mesh = plsc.VectorSubcoreMesh(core_axis_name="c", subcore_axis_name="s")
    ib = min(32, nch)
    nblk = nch // ib
    nbuf = 4

    @functools.partial(
        pl.kernel,
        out_type=jax.ShapeDtypeStruct((_NC * _NP, _H), jnp.float32),
        mesh=mesh,
        scratch_types=[
            pltpu.VMEM((ib, k), jnp.int32),
            pltpu.VMEM((ib, k), jnp.int32),
            pltpu.VMEM((k, _H), jnp.float32),
            pltpu.VMEM((k, _H), jnp.float32),
            pltpu.VMEM((k, _H), jnp.float32),
            pltpu.VMEM((k, _H), jnp.float32),
            pltpu.VMEM_SHARED((_NP, _H), jnp.float32),
            pltpu.SemaphoreType.DMA,
            pltpu.SemaphoreType.DMA,
            pltpu.SemaphoreType.DMA,
            pltpu.SemaphoreType.DMA,
        ],
    )
    def body(t_hbm, src_hbm, dst_hbm, z_hbm, out_hbm, src_v, dst_v,
             b0, b1, b2, b3, acc, s0, s1, s2, s3):
        bufs = (b0, b1, b2, b3)
        sems = (s0, s1, s2, s3)
        cid = lax.axis_index("c")
        sid = lax.axis_index("s")
        wid = cid * _NS + sid

        pltpu.sync_copy(z_hbm, acc.at[pl.ds(sid * _RPT, _RPT)])

        plsc.subcore_barrier()

        @pl.loop(0, nblk)
        def _(blk):
            base = wid * nch + blk * ib
            pltpu.async_copy(src_hbm.at[pl.ds(base, ib)], src_v, s0).wait()
            pltpu.async_copy(dst_hbm.at[pl.ds(base, ib)], dst_v, s0).wait()
            for r in range(nbuf):
                pltpu.async_copy(t_hbm.at[src_v.at[r]], bufs[r], sems[r])

            @pl.loop(0, ib // nbuf)
            def _(g):
                for r in range(nbuf):
                    j = g * nbuf + r
                    pltpu.make_async_copy(
                        t_hbm.at[src_v.at[0]], bufs[r], sems[r]).wait()
                    pltpu.sync_copy(bufs[r], acc.at[dst_v.at[j]], add=True)

                    @pl.when(j + nbuf < ib)
                    def _():
                        pltpu.async_copy(
                            t_hbm.at[src_v.at[j + nbuf]], bufs[r], sems[r])

        plsc.subcore_barrier()
        pltpu.sync_copy(acc.at[pl.ds(sid * _RPT, _RPT)],
                        out_hbm.at[pl.ds(cid * _NP + sid * _RPT, _RPT)])

    return body(t, src2, dst2, zrows)


def _histograms_sc(dst2, bat2, z16, o16):
    mesh = plsc.VectorSubcoreMesh(core_axis_name="c", subcore_axis_name="s")

    @functools.partial(
        pl.kernel,
        out_type=[jax.ShapeDtypeStruct((_NC * _NP, _H), jnp.float32),
                  jax.ShapeDtypeStruct((_NC * _CROWS, _H), jnp.float32)],
        mesh=mesh,
        scratch_types=[
            pltpu.VMEM((_NCHE, _KE), jnp.int32),
            pltpu.VMEM((_NCHP, _KP), jnp.int32),
            pltpu.VMEM((_KE, _H), jnp.float32),
            pltpu.VMEM_SHARED((_NP, _H), jnp.float32),
            pltpu.VMEM_SHARED((_CROWS, _H), jnp.float32),
            pltpu.SemaphoreType.DMA,
        ],
    )
    def body(dst_hbm, b_hbm, z_hbm, o_hbm, deg_hbm, cnt_hbm,
             dst_v, b_v, ones_v, dacc, cacc, sem):
        cid = lax.axis_index("c")
        sid = lax.axis_index("s")
        wid = cid * _NS + sid
        crpt = _CROWS // _NS

        pltpu.sync_copy(o_hbm, ones_v)
        pltpu.sync_copy(z_hbm, dacc.at[pl.ds(sid * _RPT, _RPT)])
        pltpu.sync_copy(z_hbm.at[pl.ds(0, crpt)],
                        cacc.at[pl.ds(sid * crpt, crpt)])

        plsc.subcore_barrier()

        pltpu.async_copy(dst_hbm.at[pl.ds(wid * _NCHE, _NCHE)], dst_v, sem).wait()
        pltpu.async_copy(b_hbm.at[pl.ds(wid * _NCHP, _NCHP)], b_v, sem).wait()

        @pl.loop(0, _NCHE)
        def _(j):
            pltpu.sync_copy(ones_v, dacc.at[dst_v.at[j]], add=True)

        @pl.loop(0, _NCHP)
        def _(j):
            pltpu.sync_copy(ones_v.at[pl.ds(0, _KP)], cacc.at[b_v.at[j]], add=True)

        plsc.subcore_barrier()
        pltpu.sync_copy(dacc.at[pl.ds(sid * _RPT, _RPT)],
                        deg_hbm.at[pl.ds(cid * _NP + sid * _RPT, _RPT)])
        pltpu.sync_copy(cacc.at[pl.ds(sid * crpt, crpt)],
                        cnt_hbm.at[pl.ds(cid * _CROWS + sid * crpt, crpt)])

    return body(dst2, bat2, z16, o16)


def _dinv_of(deg_ref):
    deg = deg_ref[0, :, 0:1] + deg_ref[1, :, 0:1] + 1.0
    return lax.rsqrt(deg)


def _prep_body(deg_ref, x_ref, w_ref, t0_ref):
    t0_ref[...] = _dinv_of(deg_ref) * jnp.dot(
        x_ref[...], w_ref[...], preferred_element_type=jnp.float32)


def _prep_tc(deg8, x_pad, w0):
    return pl.pallas_call(
        _prep_body,
        grid=(_NP // _RB,),
        in_specs=[pl.BlockSpec((_NC, _RB, 8), lambda i: (0, i, 0)),
                  pl.BlockSpec((_RB, _D), lambda i: (i, 0)),
                  pl.BlockSpec((_D, _H), lambda i: (0, 0))],
        out_specs=pl.BlockSpec((_RB, _H), lambda i: (i, 0)),
        out_shape=jax.ShapeDtypeStruct((_NP, _H), jnp.float32),
    )(deg8, x_pad, w0)


def _combine_mm_body(p_ref, t_ref, deg_ref, b_ref, w_ref, o_ref):
    dinv = _dinv_of(deg_ref)
    s = p_ref[0] + p_ref[1] + t_ref[...]
    h = jnp.maximum(dinv * s + b_ref[0:1, :], 0.0)
    o_ref[...] = dinv * jnp.dot(h, w_ref[...],
                                preferred_element_type=jnp.float32)


def _combine_mm(p3, t, deg8, b8, w):
    return pl.pallas_call(
        _combine_mm_body,
        grid=(_NP // _RB,),
        in_specs=[pl.BlockSpec((_NC, _RB, _H), lambda i: (0, i, 0)),
                  pl.BlockSpec((_RB, _H), lambda i: (i, 0)),
                  pl.BlockSpec((_NC, _RB, 8), lambda i: (0, i, 0)),
                  pl.BlockSpec((8, _H), lambda i: (0, 0)),
                  pl.BlockSpec((_H, _H), lambda i: (0, 0))],
        out_specs=pl.BlockSpec((_RB, _H), lambda i: (i, 0)),
        out_shape=jax.ShapeDtypeStruct((_NP, _H), jnp.float32),
    )(p3, t, deg8, b8, w)


def _combine_id_body(p_ref, t_ref, deg_ref, b_ref, o_ref):
    s = p_ref[0] + p_ref[1] + t_ref[...]
    o_ref[...] = jnp.maximum(_dinv_of(deg_ref) * s + b_ref[0:1, :], 0.0)


def _combine_id(p3, t, deg8, b8):
    return pl.pallas_call(
        _combine_id_body,
        grid=(_NP // _RB,),
        in_specs=[pl.BlockSpec((_NC, _RB, _H), lambda i: (0, i, 0)),
                  pl.BlockSpec((_RB, _H), lambda i: (i, 0)),
                  pl.BlockSpec((_NC, _RB, 8), lambda i: (0, i, 0)),
                  pl.BlockSpec((8, _H), lambda i: (0, 0))],
        out_specs=pl.BlockSpec((_RB, _H), lambda i: (i, 0)),
        out_shape=jax.ShapeDtypeStruct((_NP, _H), jnp.float32),
    )(p3, t, deg8, b8)


def _final_body(p_ref, c_ref, w0_ref, b0_ref, gam_ref, bet_ref, mu_ref,
                var_ref, w1_ref, b1_ref, o_ref):
    pooled = p_ref[0] + p_ref[1]
    cnt = c_ref[0, :, 0:1] + c_ref[1, :, 0:1]
    mean = pooled / jnp.maximum(cnt, 1.0)
    z = jnp.dot(mean, w0_ref[...], preferred_element_type=jnp.float32) + b0_ref[0:1, :]
    z = (z - mu_ref[0:1, :]) * lax.rsqrt(var_ref[0:1, :] + 1e-5) * gam_ref[0:1, :] + bet_ref[0:1, :]
    z = jnp.maximum(z, 0.0)
    out = jnp.dot(z, w1_ref[...], preferred_element_type=jnp.float32) + b1_ref[0:1, :]
    m = jnp.max(out, axis=1, keepdims=True)
    lse = jnp.log(jnp.sum(jnp.exp(out - m), axis=1, keepdims=True)) + m
    o_ref[...] = out - lse


def _final_tc(pp3, cnt3, w0, b0, gam, bet, mu, var, w1, b1):
    return pl.pallas_call(
        _final_body,
        out_shape=jax.ShapeDtypeStruct((_G, _C), jnp.float32),
    )(pp3, cnt3, w0, b0, gam, bet, mu, var, w1, b1)


def kernel(x, edge_index, batch, conv_W0, conv_b0, conv_W1, conv_b1,
           conv_W2, conv_b2, mlp_W0, mlp_b0, bn_gamma, bn_beta, bn_mean,
           bn_var, mlp_W1, mlp_b1):
    src = edge_index[0].astype(jnp.int32)
    dst = edge_index[1].astype(jnp.int32)
    pad_e = _EP - _E
    pad_src = jnp.arange(pad_e, dtype=jnp.int32) % _N
    src_p = jnp.concatenate([src, pad_src]).reshape(_NW * _NCHE, _KE)
    pad_dst = _N + jnp.arange(pad_e, dtype=jnp.int32) % (_NP - _N)
    dst_p = jnp.concatenate([dst, pad_dst]).reshape(_NW * _NCHE, _KE)

    bat = batch.astype(jnp.int32)
    pad_bat = _G + jnp.arange(_PP - _N, dtype=jnp.int32) % (_CROWS - _G)
    bat_p = jnp.concatenate([bat, pad_bat]).reshape(_NW * _NCHP, _KP)
    pool_src = jnp.concatenate(
        [jnp.arange(_N, dtype=jnp.int32),
         jnp.arange(_PP - _N, dtype=jnp.int32) % _N]).reshape(_NW * _NCHP, _KP)

    x_pad = jnp.pad(x, ((0, _NP - _N), (0, 0)))

    zrows = jnp.zeros((_RPT, _H), jnp.float32)
    o128 = jnp.ones((_KE, _H), jnp.float32)

    deg_f, cnt_f = _histograms_sc(dst_p, bat_p, zrows, o128)
    deg8 = deg_f.reshape(_NC, _NP, _H)[:, :, :8]
    cnt3 = cnt_f.reshape(_NC, _CROWS, _H)[:, :_G, :8]

    t0 = _prep_tc(deg8, x_pad, conv_W0)

    def b8(v):
        return jnp.tile(v[None, :], (8, 1))

    p0 = _segment_rows_sc(t0, src_p, dst_p, zrows, _NCHE, _KE).reshape(_NC, _NP, _H)
    t1 = _combine_mm(p0, t0, deg8, b8(conv_b0), conv_W1)
    p1 = _segment_rows_sc(t1, src_p, dst_p, zrows, _NCHE, _KE).reshape(_NC, _NP, _H)
    t2 = _combine_mm(p1, t1, deg8, b8(conv_b1), conv_W2)
    p2 = _segment_rows_sc(t2, src_p, dst_p, zrows, _NCHE, _KE).reshape(_NC, _NP, _H)
    h3 = _combine_id(p2, t2, deg8, b8(conv_b2))

    pp3 = _segment_rows_sc(h3, pool_src, bat_p, zrows, _NCHP, _KP)
    pp3 = pp3.reshape(_NC, _NP, _H)[:, :_G]

    return _final_tc(pp3, cnt3, mlp_W0, b8(mlp_b0), b8(bn_gamma),
                     b8(bn_beta), b8(bn_mean), b8(bn_var), mlp_W1, b8(mlp_b1))
